# baseline (speedup 1.0000x reference)
import functools

import jax
import jax.numpy as jnp
from jax import lax
from jax.experimental import pallas as pl
from jax.experimental.pallas import tpu as pltpu
from jax.experimental.pallas import tpu_sc as plsc

N = 10000
E = 320000
G = 128
OUT = 16

NC = 2
NS = 16
CHUNK = 128
GROUP = 16
NG0 = 5
NG = 10
CPT0 = NG0 * GROUP
CPT = NG * GROUP
N_PAD = 10240
RPT = N_PAD // NS

R = 512
NSTEP = N_PAD // R


def _sc_mesh():
    return plsc.VectorSubcoreMesh(
        core_axis_name="c", subcore_axis_name="s", num_cores=NC,
        num_subcores=NS)


@functools.cache
def _make_spmm(ngroups, edge_split):
    out_type = jax.ShapeDtypeStruct((NC, N_PAD, 128), jnp.float32)
    scratch = [
        pltpu.VMEM((2, GROUP, CHUNK), jnp.int32),
        pltpu.VMEM((2, GROUP, CHUNK), jnp.int32),
        pltpu.VMEM((2, CHUNK, 128), jnp.float32),
        pltpu.VMEM_SHARED((N_PAD, 128), jnp.float32),
        pltpu.SemaphoreType.DMA,
        pltpu.SemaphoreType.DMA,
        pltpu.SemaphoreType.DMA,
        pltpu.SemaphoreType.DMA,
        pltpu.SemaphoreType.DMA,
        pltpu.SemaphoreType.DMA,
        pltpu.SemaphoreType.DMA,
    ]

    def body(tab, srcr, dstr, zrow, z_out, s_src, s_dst, rows, zsh,
             g0a, g0b, g1a, g1b, ss0, ss1, sp):
        cid = lax.axis_index("c")
        sid = lax.axis_index("s")
        base = sid * RPT

        def stage(g, p, wait):
            off = g * GROUP
            if edge_split:
                dsrc = dstr.at[cid, sid, pl.ds(off, GROUP)]
            else:
                dsrc = dstr.at[sid, pl.ds(off, GROUP)]
            c0 = pltpu.async_copy(srcr.at[cid, sid, pl.ds(off, GROUP)],
                                  s_src.at[p], sp)
            c1 = pltpu.async_copy(dsrc, s_dst.at[p], sp)
            if wait:
                c0.wait()
                c1.wait()
            return c0, c1

        def wait_stage(g, p):
            off = g * GROUP
            if edge_split:
                dsrc = dstr.at[cid, sid, pl.ds(off, GROUP)]
            else:
                dsrc = dstr.at[sid, pl.ds(off, GROUP)]
            pltpu.make_async_copy(srcr.at[cid, sid, pl.ds(off, GROUP)],
                                  s_src.at[p], sp).wait()
            pltpu.make_async_copy(dsrc, s_dst.at[p], sp).wait()

        pltpu.sync_copy(zrow, rows.at[0])
        for k in range(RPT // CHUNK):
            pltpu.sync_copy(rows.at[0],
                            zsh.at[pl.ds(base + k * CHUNK, CHUNK)])

        stage(0, 0, True)

        plsc.subcore_barrier()

        H = CHUNK // 2

        def fire(p, j, b):
            sa, sb = (g0a, g0b) if b == 0 else (g1a, g1b)
            ca = pltpu.async_copy(tab.at[s_src.at[p, j, pl.ds(0, H)]],
                                  rows.at[b, pl.ds(0, H)], sa)
            cb = pltpu.async_copy(tab.at[s_src.at[p, j, pl.ds(H, H)]],
                                  rows.at[b, pl.ds(H, H)], sb)
            return ca, cb

        fire(0, 0, 0)
        fire(0, 1, 1)

        def wait_buf(b):
            sa, sb = (g0a, g0b) if b == 0 else (g1a, g1b)
            pltpu.make_async_copy(tab.at[pl.ds(0, H)],
                                  rows.at[b, pl.ds(0, H)], sa).wait()
            pltpu.make_async_copy(tab.at[pl.ds(0, H)],
                                  rows.at[b, pl.ds(H, H)], sb).wait()

        @pl.loop(0, ngroups)
        def _(g):
            p = g % 2

            @pl.when(g + 1 < ngroups)
            def _():
                stage(g + 1, 1 - p, False)

            for j in range(0, GROUP, 2):
                wait_buf(0)
                sc0 = pltpu.async_copy(rows.at[0], zsh.at[s_dst.at[p, j]],
                                       ss0, add=True)
                wait_buf(1)
                sc1 = pltpu.async_copy(rows.at[1],
                                       zsh.at[s_dst.at[p, j + 1]], ss1,
                                       add=True)
                sc0.wait()
                sc1.wait()
                if j + 2 < GROUP:
                    fire(p, j + 2, 0)
                    fire(p, j + 3, 1)
                else:
                    @pl.when(g + 1 < ngroups)
                    def _():
                        wait_stage(g + 1, 1 - p)
                        fire(1 - p, 0, 0)
                        fire(1 - p, 1, 1)

        plsc.subcore_barrier()

        pltpu.sync_copy(zsh.at[pl.ds(base, RPT)],
                        z_out.at[cid, pl.ds(base, RPT)])

    return pl.kernel(body, out_type=out_type, mesh=_sc_mesh(),
                     scratch_types=scratch)


@functools.cache
def _make_deg():
    out_type = jax.ShapeDtypeStruct((NC, N_PAD, 128), jnp.float32)
    scratch = [
        pltpu.VMEM((GROUP, CHUNK), jnp.int32),
        pltpu.VMEM((CHUNK, 128), jnp.float32),
        pltpu.VMEM((CHUNK, 128), jnp.float32),
        pltpu.VMEM_SHARED((N_PAD, 128), jnp.float32),
    ]

    def body(dstr, zrow, orow, deg_out, s_dst, ones_b, zb, degsh):
        cid = lax.axis_index("c")
        sid = lax.axis_index("s")
        base = sid * RPT

        pltpu.sync_copy(orow, ones_b)
        pltpu.sync_copy(zrow, zb)
        for k in range(RPT // CHUNK):
            pltpu.sync_copy(zb, degsh.at[pl.ds(base + k * CHUNK, CHUNK)])

        plsc.subcore_barrier()

        @pl.loop(0, NG0)
        def _(g):
            pltpu.sync_copy(dstr.at[cid, sid, pl.ds(g * GROUP, GROUP)],
                            s_dst)
            for j in range(GROUP):
                pltpu.sync_copy(ones_b, degsh.at[s_dst.at[j]], add=True)

        plsc.subcore_barrier()

        pltpu.sync_copy(degsh.at[pl.ds(base, RPT)],
                        deg_out.at[cid, pl.ds(base, RPT)])

    return pl.kernel(body, out_type=out_type, mesh=_sc_mesh(),
                     scratch_types=scratch)


def _dense_mid_body(z, h, deg, w, b1f, csh, o, *, split_h):
    i = pl.program_id(0)
    dcol = deg[:, 0:1]
    if split_h:
        x = jnp.concatenate(
            [z[0], z[1], h[0], h[1], dcol * h[0], dcol * h[1]], axis=1)
    else:
        x = jnp.concatenate([z[0], z[1], h[...], dcol * h[...]], axis=1)
    y = jnp.dot(x, w[...], preferred_element_type=jnp.float32)
    y = y + dcol * b1f[...] + csh[...]
    y = jnp.maximum(y, 0.0)
    rowid = i * R + lax.broadcasted_iota(jnp.int32, (R, 1), 0)
    y = jnp.where(rowid < N, y, 0.0)
    o[0] = y[:, :128]
    o[1] = y[:, 128:]


def _make_dense_mid(split_h):
    h_spec = (pl.BlockSpec((2, R, 128), lambda i: (0, i, 0)) if split_h
              else pl.BlockSpec((R, 128), lambda i: (i, 0)))
    k = 768 if split_h else 512
    return pl.pallas_call(
        functools.partial(_dense_mid_body, split_h=split_h),
        grid=(NSTEP,),
        in_specs=[
            pl.BlockSpec((2, R, 128), lambda i: (0, i, 0)),
            h_spec,
            pl.BlockSpec((R, 128), lambda i: (i, 0)),
            pl.BlockSpec((k, 256), lambda i: (0, 0)),
            pl.BlockSpec((1, 256), lambda i: (0, 0)),
            pl.BlockSpec((1, 256), lambda i: (0, 0)),
        ],
        out_specs=pl.BlockSpec((2, R, 128), lambda i: (0, i, 0)),
        out_shape=jax.ShapeDtypeStruct((2, N_PAD, 128), jnp.float32),
    )


def _dense_last_body(z, h, deg, bat, w, b1f, b3r, scl, shf, fcw, fcb, o,
                     sums, cnt):
    i = pl.program_id(0)
    dcol = deg[:, 0:1]
    x = jnp.concatenate(
        [z[0], z[1], h[0], h[1], dcol * h[0], dcol * h[1]], axis=1)
    y = jnp.dot(x, w[...], preferred_element_type=jnp.float32)
    y = y + dcol * b1f[...] + b3r[...]
    y = jnp.maximum(y, 0.0)
    y = y * scl[...] + shf[...]
    rowid = i * R + lax.broadcasted_iota(jnp.int32, (R, 1), 0)
    y = jnp.where(rowid < N, y, 0.0)

    bcol = bat[:, 0:1]
    gid = lax.broadcasted_iota(jnp.int32, (1, G), 1).astype(jnp.float32)
    oneh = (bcol == gid).astype(jnp.float32)

    @pl.when(i == 0)
    def _():
        sums[...] = jnp.zeros_like(sums)
        cnt[...] = jnp.zeros_like(cnt)

    sums[...] += lax.dot_general(
        oneh, y, (((0,), (0,)), ((), ())),
        preferred_element_type=jnp.float32)
    cnt[...] += lax.dot_general(
        oneh, jnp.ones((R, 8), jnp.float32), (((0,), (0,)), ((), ())),
        preferred_element_type=jnp.float32)

    @pl.when(i == NSTEP - 1)
    def _():
        pooled = sums[...] / jnp.maximum(cnt[:, 0:1], 1.0)
        o[...] = jnp.dot(pooled, fcw[...],
                         preferred_element_type=jnp.float32) + fcb[...]


_dense_last = pl.pallas_call(
    _dense_last_body,
    grid=(NSTEP,),
    in_specs=[
        pl.BlockSpec((2, R, 128), lambda i: (0, i, 0)),
        pl.BlockSpec((2, R, 128), lambda i: (0, i, 0)),
        pl.BlockSpec((R, 128), lambda i: (i, 0)),
        pl.BlockSpec((R, 8), lambda i: (i, 0)),
        pl.BlockSpec((768, 256), lambda i: (0, 0)),
        pl.BlockSpec((1, 256), lambda i: (0, 0)),
        pl.BlockSpec((1, 256), lambda i: (0, 0)),
        pl.BlockSpec((1, 256), lambda i: (0, 0)),
        pl.BlockSpec((1, 256), lambda i: (0, 0)),
        pl.BlockSpec((256, OUT), lambda i: (0, 0)),
        pl.BlockSpec((1, OUT), lambda i: (0, 0)),
    ],
    out_specs=pl.BlockSpec((G, OUT), lambda i: (0, 0)),
    out_shape=jax.ShapeDtypeStruct((G, OUT), jnp.float32),
    scratch_shapes=[
        pltpu.VMEM((G, 256), jnp.float32),
        pltpu.VMEM((G, 8), jnp.float32),
    ],
)

_dense0 = _make_dense_mid(False)
_dense1 = _make_dense_mid(True)


def _affine(g, b, m, v):
    scale = g * lax.rsqrt(v + 1e-5)
    return scale, b - m * scale


def kernel(x, edge_index, batch,
           W1_0, W2_0, W3_0, b1_0, b3_0, bn_g_0, bn_b_0, bn_m_0, bn_v_0,
           W1_1, W2_1, W3_1, b1_1, b3_1, bn_g_1, bn_b_1, bn_m_1, bn_v_1,
           W1_2, W2_2, W3_2, b1_2, b3_2, bn_g_2, bn_b_2, bn_m_2, bn_v_2,
           fc_W, fc_b):
    src = edge_index[0]
    dst = edge_index[1]

    eh = E // 2
    padh = NS * CPT0 * CHUNK - eh
    fillh = jnp.full((padh,), N, jnp.int32)
    src_l0 = jnp.stack([jnp.concatenate([src[:eh], fillh]),
                        jnp.concatenate([src[eh:], fillh])])
    src_l0 = src_l0.reshape(NC, NS, CPT0, CHUNK)
    dst_l0 = jnp.stack([jnp.concatenate([dst[:eh], fillh]),
                        jnp.concatenate([dst[eh:], fillh])])
    dst_l0 = dst_l0.reshape(NC, NS, CPT0, CHUNK)

    pad = NS * CPT * CHUNK - E
    fill = jnp.full((pad,), N, jnp.int32)
    srcp = jnp.concatenate([src, fill])
    dstp = jnp.concatenate([dst, fill])
    src2 = jnp.stack([srcp, srcp + N_PAD]).reshape(NC, NS, CPT, CHUNK)
    dstr = dstp.reshape(NS, CPT, CHUNK)

    xp = jnp.zeros((N_PAD, 128), jnp.float32).at[:N].set(x)

    batf = jnp.full((N_PAD,), float(G), jnp.float32).at[:N].set(
        batch.astype(jnp.float32))
    batf = jnp.broadcast_to(batf[:, None], (N_PAD, 8))

    zr128 = jnp.zeros((CHUNK, 128), jnp.float32)
    o128 = jnp.ones((CHUNK, 128), jnp.float32)

    def prep_mid(W1, W2, W3, b1, b3, g, b, m, v, dup_w1):
        scale, shift = _affine(g, b, m, v)
        blocks = ([W1, W1, W3, -W2] if dup_w1 else [W1, W3, -W2])
        w = jnp.concatenate(blocks, axis=0) * scale[None, :]
        return w, (b1 * scale)[None, :], (b3 * scale + shift)[None, :]

    w0, b1f0, csh0 = prep_mid(W1_0, W2_0, W3_0, b1_0, b3_0,
                              bn_g_0, bn_b_0, bn_m_0, bn_v_0, True)
    w1, b1f1, csh1 = prep_mid(W1_1, W2_1, W3_1, b1_1, b3_1,
                              bn_g_1, bn_b_1, bn_m_1, bn_v_1, False)
    w2 = jnp.concatenate([W1_2, W3_2, -W2_2], axis=0)
    scale2, shift2 = _affine(bn_g_2, bn_b_2, bn_m_2, bn_v_2)

    _spmm_l0 = _make_spmm(NG0, True)
    _spmm = _make_spmm(NG, False)
    _deg = _make_deg()

    deg2 = _deg(dst_l0, zr128, o128)
    deg = deg2[0] + deg2[1]
    z = _spmm_l0(xp, src_l0, dst_l0, zr128)
    h = _dense0(z, xp, deg, w0, b1f0, csh0)
    z = _spmm(h.reshape(NC * N_PAD, 128), src2, dstr, zr128)
    h = _dense1(z, h, deg, w1, b1f1, csh1)
    z = _spmm(h.reshape(NC * N_PAD, 128), src2, dstr, zr128)
    out = _dense_last(z, h, deg, batf, w2, b1_2[None, :],
                      b3_2[None, :], scale2[None, :], shift2[None, :],
                      fc_W, fc_b[None, :])
    return out


# speedup vs baseline: 1.0365x; 1.0365x over previous
import functools

import jax
import jax.numpy as jnp
from jax import lax
from jax.experimental import pallas as pl
from jax.experimental.pallas import tpu as pltpu
from jax.experimental.pallas import tpu_sc as plsc

N = 10000
E = 320000
G = 128
OUT = 16

NC = 2
NS = 16
CHUNK = 128
GROUP = 16
NG0 = 5
NG = 10
CPT0 = NG0 * GROUP
CPT = NG * GROUP
N_PAD = 10240
RPT = N_PAD // NS

R = 512
NSTEP = N_PAD // R


def _sc_mesh():
    return plsc.VectorSubcoreMesh(
        core_axis_name="c", subcore_axis_name="s", num_cores=NC,
        num_subcores=NS)


@functools.cache
def _make_spmm(ngroups, edge_split):
    out_type = jax.ShapeDtypeStruct((NC, N_PAD, 128), jnp.float32)
    scratch = [
        pltpu.VMEM((2, GROUP, CHUNK), jnp.int32),
        pltpu.VMEM((2, GROUP, CHUNK), jnp.int32),
        pltpu.VMEM((2, CHUNK, 128), jnp.float32),
        pltpu.VMEM_SHARED((N_PAD, 128), jnp.float32),
        pltpu.SemaphoreType.DMA,
        pltpu.SemaphoreType.DMA,
        pltpu.SemaphoreType.DMA,
        pltpu.SemaphoreType.DMA,
        pltpu.SemaphoreType.DMA,
        pltpu.SemaphoreType.DMA,
        pltpu.SemaphoreType.DMA,
    ]

    def body(tab, srcr, dstr, zrow, z_out, s_src, s_dst, rows, zsh,
             g0a, g0b, g1a, g1b, ss0, ss1, sp):
        cid = lax.axis_index("c")
        sid = lax.axis_index("s")
        base = sid * RPT

        def stage(g, p, wait):
            off = g * GROUP
            if edge_split:
                dsrc = dstr.at[cid, sid, pl.ds(off, GROUP)]
            else:
                dsrc = dstr.at[sid, pl.ds(off, GROUP)]
            c0 = pltpu.async_copy(srcr.at[cid, sid, pl.ds(off, GROUP)],
                                  s_src.at[p], sp)
            c1 = pltpu.async_copy(dsrc, s_dst.at[p], sp)
            if wait:
                c0.wait()
                c1.wait()
            return c0, c1

        def wait_stage(g, p):
            off = g * GROUP
            if edge_split:
                dsrc = dstr.at[cid, sid, pl.ds(off, GROUP)]
            else:
                dsrc = dstr.at[sid, pl.ds(off, GROUP)]
            pltpu.make_async_copy(srcr.at[cid, sid, pl.ds(off, GROUP)],
                                  s_src.at[p], sp).wait()
            pltpu.make_async_copy(dsrc, s_dst.at[p], sp).wait()

        pltpu.sync_copy(zrow, rows.at[0])
        for k in range(RPT // CHUNK):
            pltpu.sync_copy(rows.at[0],
                            zsh.at[pl.ds(base + k * CHUNK, CHUNK)])

        stage(0, 0, True)

        plsc.subcore_barrier()

        H = CHUNK // 2

        def fire(p, j, b):
            sa, sb = (g0a, g0b) if b == 0 else (g1a, g1b)
            ca = pltpu.async_copy(tab.at[s_src.at[p, j, pl.ds(0, H)]],
                                  rows.at[b, pl.ds(0, H)], sa)
            cb = pltpu.async_copy(tab.at[s_src.at[p, j, pl.ds(H, H)]],
                                  rows.at[b, pl.ds(H, H)], sb)
            return ca, cb

        fire(0, 0, 0)
        fire(0, 1, 1)

        def wait_buf(b):
            sa, sb = (g0a, g0b) if b == 0 else (g1a, g1b)
            pltpu.make_async_copy(tab.at[pl.ds(0, H)],
                                  rows.at[b, pl.ds(0, H)], sa).wait()
            pltpu.make_async_copy(tab.at[pl.ds(0, H)],
                                  rows.at[b, pl.ds(H, H)], sb).wait()

        @pl.loop(0, ngroups)
        def _(g):
            p = g % 2

            @pl.when(g + 1 < ngroups)
            def _():
                stage(g + 1, 1 - p, False)

            for j in range(0, GROUP, 2):
                wait_buf(0)
                sc0 = pltpu.async_copy(rows.at[0], zsh.at[s_dst.at[p, j]],
                                       ss0, add=True)
                wait_buf(1)
                sc1 = pltpu.async_copy(rows.at[1],
                                       zsh.at[s_dst.at[p, j + 1]], ss1,
                                       add=True)
                sc0.wait()
                sc1.wait()
                if j + 2 < GROUP:
                    fire(p, j + 2, 0)
                    fire(p, j + 3, 1)
                else:
                    @pl.when(g + 1 < ngroups)
                    def _():
                        wait_stage(g + 1, 1 - p)
                        fire(1 - p, 0, 0)
                        fire(1 - p, 1, 1)

        plsc.subcore_barrier()

        pltpu.sync_copy(zsh.at[pl.ds(base, RPT)],
                        z_out.at[cid, pl.ds(base, RPT)])

    return pl.kernel(body, out_type=out_type, mesh=_sc_mesh(),
                     scratch_types=scratch)


@functools.cache
def _make_deg():
    out_type = jax.ShapeDtypeStruct((NC, N_PAD, 128), jnp.float32)
    scratch = [
        pltpu.VMEM((GROUP, CHUNK), jnp.int32),
        pltpu.VMEM((CHUNK, 128), jnp.float32),
        pltpu.VMEM((CHUNK, 128), jnp.float32),
        pltpu.VMEM_SHARED((N_PAD, 128), jnp.float32),
    ]

    def body(dstr, zrow, orow, deg_out, s_dst, ones_b, zb, degsh):
        cid = lax.axis_index("c")
        sid = lax.axis_index("s")
        base = sid * RPT

        pltpu.sync_copy(orow, ones_b)
        pltpu.sync_copy(zrow, zb)
        for k in range(RPT // CHUNK):
            pltpu.sync_copy(zb, degsh.at[pl.ds(base + k * CHUNK, CHUNK)])

        plsc.subcore_barrier()

        @pl.loop(0, NG0)
        def _(g):
            pltpu.sync_copy(dstr.at[cid, sid, pl.ds(g * GROUP, GROUP)],
                            s_dst)
            for j in range(GROUP):
                pltpu.sync_copy(ones_b, degsh.at[s_dst.at[j]], add=True)

        plsc.subcore_barrier()

        pltpu.sync_copy(degsh.at[pl.ds(base, RPT)],
                        deg_out.at[cid, pl.ds(base, RPT)])

    return pl.kernel(body, out_type=out_type, mesh=_sc_mesh(),
                     scratch_types=scratch)


def _dense_mid_body(z, h, deg, w, b1f, csh, o, *, split_h):
    i = pl.program_id(0)
    dcol = deg[0][:, 0:1] + deg[1][:, 0:1]
    if split_h:
        x = jnp.concatenate(
            [z[0], z[1], h[0], h[1], dcol * h[0], dcol * h[1]], axis=1)
    else:
        x = jnp.concatenate([z[0], z[1], h[...], dcol * h[...]], axis=1)
    y = jnp.dot(x, w[...], preferred_element_type=jnp.float32)
    y = y + dcol * b1f[...] + csh[...]
    y = jnp.maximum(y, 0.0)
    rowid = i * R + lax.broadcasted_iota(jnp.int32, (R, 1), 0)
    y = jnp.where(rowid < N, y, 0.0)
    o[0] = y[:, :128]
    o[1] = y[:, 128:]


def _make_dense_mid(split_h):
    h_spec = (pl.BlockSpec((2, R, 128), lambda i: (0, i, 0)) if split_h
              else pl.BlockSpec((R, 128), lambda i: (i, 0)))
    k = 768 if split_h else 512
    return pl.pallas_call(
        functools.partial(_dense_mid_body, split_h=split_h),
        grid=(NSTEP,),
        in_specs=[
            pl.BlockSpec((2, R, 128), lambda i: (0, i, 0)),
            h_spec,
            pl.BlockSpec((2, R, 128), lambda i: (0, i, 0)),
            pl.BlockSpec((k, 256), lambda i: (0, 0)),
            pl.BlockSpec((1, 256), lambda i: (0, 0)),
            pl.BlockSpec((1, 256), lambda i: (0, 0)),
        ],
        out_specs=pl.BlockSpec((2, R, 128), lambda i: (0, i, 0)),
        out_shape=jax.ShapeDtypeStruct((2, N_PAD, 128), jnp.float32),
    )


def _dense_last_body(z, h, deg, bat, w, b1f, b3r, scl, shf, fcw, fcb, o,
                     sums, cnt):
    i = pl.program_id(0)
    dcol = deg[0][:, 0:1] + deg[1][:, 0:1]
    x = jnp.concatenate(
        [z[0], z[1], h[0], h[1], dcol * h[0], dcol * h[1]], axis=1)
    y = jnp.dot(x, w[...], preferred_element_type=jnp.float32)
    y = y + dcol * b1f[...] + b3r[...]
    y = jnp.maximum(y, 0.0)
    y = y * scl[...] + shf[...]
    rowid = i * R + lax.broadcasted_iota(jnp.int32, (R, 1), 0)
    y = jnp.where(rowid < N, y, 0.0)

    bcol = bat[:, 0:1]
    gid = lax.broadcasted_iota(jnp.int32, (1, G), 1).astype(jnp.float32)
    oneh = (bcol == gid).astype(jnp.float32)

    @pl.when(i == 0)
    def _():
        sums[...] = jnp.zeros_like(sums)
        cnt[...] = jnp.zeros_like(cnt)

    sums[...] += lax.dot_general(
        oneh, y, (((0,), (0,)), ((), ())),
        preferred_element_type=jnp.float32)
    cnt[...] += lax.dot_general(
        oneh, jnp.ones((R, 8), jnp.float32), (((0,), (0,)), ((), ())),
        preferred_element_type=jnp.float32)

    @pl.when(i == NSTEP - 1)
    def _():
        pooled = sums[...] / jnp.maximum(cnt[:, 0:1], 1.0)
        o[...] = jnp.dot(pooled, fcw[...],
                         preferred_element_type=jnp.float32) + fcb[...]


_dense_last = pl.pallas_call(
    _dense_last_body,
    grid=(NSTEP,),
    in_specs=[
        pl.BlockSpec((2, R, 128), lambda i: (0, i, 0)),
        pl.BlockSpec((2, R, 128), lambda i: (0, i, 0)),
        pl.BlockSpec((2, R, 128), lambda i: (0, i, 0)),
        pl.BlockSpec((R, 8), lambda i: (i, 0)),
        pl.BlockSpec((768, 256), lambda i: (0, 0)),
        pl.BlockSpec((1, 256), lambda i: (0, 0)),
        pl.BlockSpec((1, 256), lambda i: (0, 0)),
        pl.BlockSpec((1, 256), lambda i: (0, 0)),
        pl.BlockSpec((1, 256), lambda i: (0, 0)),
        pl.BlockSpec((256, OUT), lambda i: (0, 0)),
        pl.BlockSpec((1, OUT), lambda i: (0, 0)),
    ],
    out_specs=pl.BlockSpec((G, OUT), lambda i: (0, 0)),
    out_shape=jax.ShapeDtypeStruct((G, OUT), jnp.float32),
    scratch_shapes=[
        pltpu.VMEM((G, 256), jnp.float32),
        pltpu.VMEM((G, 8), jnp.float32),
    ],
)

_dense0 = _make_dense_mid(False)
_dense1 = _make_dense_mid(True)


def _affine(g, b, m, v):
    scale = g * lax.rsqrt(v + 1e-5)
    return scale, b - m * scale


def kernel(x, edge_index, batch,
           W1_0, W2_0, W3_0, b1_0, b3_0, bn_g_0, bn_b_0, bn_m_0, bn_v_0,
           W1_1, W2_1, W3_1, b1_1, b3_1, bn_g_1, bn_b_1, bn_m_1, bn_v_1,
           W1_2, W2_2, W3_2, b1_2, b3_2, bn_g_2, bn_b_2, bn_m_2, bn_v_2,
           fc_W, fc_b):
    src = edge_index[0]
    dst = edge_index[1]

    eh = E // 2
    padh = NS * CPT0 * CHUNK - eh
    fillh = jnp.full((padh,), N, jnp.int32)
    src_l0 = jnp.stack([jnp.concatenate([src[:eh], fillh]),
                        jnp.concatenate([src[eh:], fillh])])
    src_l0 = src_l0.reshape(NC, NS, CPT0, CHUNK)
    dst_l0 = jnp.stack([jnp.concatenate([dst[:eh], fillh]),
                        jnp.concatenate([dst[eh:], fillh])])
    dst_l0 = dst_l0.reshape(NC, NS, CPT0, CHUNK)

    pad = NS * CPT * CHUNK - E
    fill = jnp.full((pad,), N, jnp.int32)
    srcp = jnp.concatenate([src, fill])
    dstp = jnp.concatenate([dst, fill])
    src2 = jnp.stack([srcp, srcp + N_PAD]).reshape(NC, NS, CPT, CHUNK)
    dstr = dstp.reshape(NS, CPT, CHUNK)

    xp = jnp.zeros((N_PAD, 128), jnp.float32).at[:N].set(x)

    batf = jnp.full((N_PAD,), float(G), jnp.float32).at[:N].set(
        batch.astype(jnp.float32))
    batf = jnp.broadcast_to(batf[:, None], (N_PAD, 8))

    zr128 = jnp.zeros((CHUNK, 128), jnp.float32)
    o128 = jnp.ones((CHUNK, 128), jnp.float32)

    def prep_mid(W1, W2, W3, b1, b3, g, b, m, v, dup_w1):
        scale, shift = _affine(g, b, m, v)
        blocks = ([W1, W1, W3, -W2] if dup_w1 else [W1, W3, -W2])
        w = jnp.concatenate(blocks, axis=0) * scale[None, :]
        return w, (b1 * scale)[None, :], (b3 * scale + shift)[None, :]

    w0, b1f0, csh0 = prep_mid(W1_0, W2_0, W3_0, b1_0, b3_0,
                              bn_g_0, bn_b_0, bn_m_0, bn_v_0, True)
    w1, b1f1, csh1 = prep_mid(W1_1, W2_1, W3_1, b1_1, b3_1,
                              bn_g_1, bn_b_1, bn_m_1, bn_v_1, False)
    w2 = jnp.concatenate([W1_2, W3_2, -W2_2], axis=0)
    scale2, shift2 = _affine(bn_g_2, bn_b_2, bn_m_2, bn_v_2)

    _spmm_l0 = _make_spmm(NG0, True)
    _spmm = _make_spmm(NG, False)
    _deg = _make_deg()

    deg = _deg(dst_l0, zr128, o128)
    z = _spmm_l0(xp, src_l0, dst_l0, zr128)
    h = _dense0(z, xp, deg, w0, b1f0, csh0)
    z = _spmm(h.reshape(NC * N_PAD, 128), src2, dstr, zr128)
    h = _dense1(z, h, deg, w1, b1f1, csh1)
    z = _spmm(h.reshape(NC * N_PAD, 128), src2, dstr, zr128)
    out = _dense_last(z, h, deg, batf, w2, b1_2[None, :],
                      b3_2[None, :], scale2[None, :], shift2[None, :],
                      fc_W, fc_b[None, :])
    return out
